# baseline (speedup 1.0000x reference)
import jax
import jax.numpy as jnp
from jax import lax
from jax.experimental import pallas as pl
from jax.experimental.pallas import tpu as pltpu
from jax.experimental.pallas import tpu_sc as plsc

E, N, D, R = 160000, 10000, 256, 237
NC, NS, L = 2, 16, 16
NW = NC * NS
NPAD = 10240
NPT = NPAD // NS
EP = 163840
EPW = EP // NW
EPT = EP // NS
CH1 = 64
CH2 = 1024
SB2 = 128
CH4 = 64
DH = D // 2
NEG_INF = float("-inf")


def _mesh():
    return plsc.VectorSubcoreMesh(core_axis_name="c", subcore_axis_name="s")


SUP1 = 8
SUPE1 = SUP1 * CH1
W0T = 4096
W1T = (EP - NS * W0T) // NS


def _k1_body(ent_hbm, rel_hbm, src_hbm, dst_hbm, rid_hbm,
             scores_hbm, pmax_hbm,
             srcv, dstv, ridv, scout,
             ss0, ss1, ds0, ds1, rs0, rs1,
             rowss0, rowss1, rowsd0, rowsd1, rowsr0, rowsr1,
             smax_local, tmpv, red_v, smax_shared, sem0, sem1):
    c = lax.axis_index("c")
    s = lax.axis_index("s")
    base = jnp.where(c == 0, s * W0T, NS * W0T + s * W1T)
    nsup = jnp.where(c == 0, W0T // SUPE1, W1T // SUPE1)
    lanes = lax.iota(jnp.int32, L)
    sss, dss, rss = [ss0, ss1], [ds0, ds1], [rs0, rs1]
    rows_ss, rows_ds, rows_rs = [rowss0, rowss1], [rowsd0, rowsd1], \
        [rowsr0, rowsr1]
    sems = [sem0, sem1]

    neg = jnp.full((L,), NEG_INF, jnp.float32)

    def init_b(i, carry):
        smax_local[pl.ds(i * L, L)] = neg
        return carry

    lax.fori_loop(0, NPAD // L, init_b, 0)

    def build_idx(ch, sl_i):
        for k in range(CH1 // L):
            sl = pl.ds(k * L, L)
            sss[sl_i][sl] = srcv[pl.ds(ch * CH1 + k * L, L)]
            dss[sl_i][sl] = dstv[pl.ds(ch * CH1 + k * L, L)]
            rss[sl_i][sl] = ridv[pl.ds(ch * CH1 + k * L, L)]

    def issue(sl_i):
        cp1 = pltpu.async_copy(ent_hbm.at[sss[sl_i]], rows_ss[sl_i],
                               sems[sl_i])
        cp2 = pltpu.async_copy(ent_hbm.at[dss[sl_i]], rows_ds[sl_i],
                               sems[sl_i])
        cp3 = pltpu.async_copy(rel_hbm.at[rss[sl_i]], rows_rs[sl_i],
                               sems[sl_i])
        return cp1, cp2, cp3

    def sup_body(sup, carry):
        soff = base + sup * SUPE1
        pltpu.sync_copy(src_hbm.at[pl.ds(soff, SUPE1)], srcv)
        pltpu.sync_copy(dst_hbm.at[pl.ds(soff, SUPE1)], dstv)
        pltpu.sync_copy(rid_hbm.at[pl.ds(soff, SUPE1)], ridv)

        for ch in range(SUP1):
            sl_i = ch % 2
            if ch == 0:
                build_idx(0, 0)
                cps = issue(0)
            else:
                cps = nxt
            if ch + 1 < SUP1:
                build_idx(ch + 1, (ch + 1) % 2)
                nxt = issue((ch + 1) % 2)
            cps[0].wait()
            cps[1].wait()
            cps[2].wait()
            rows_s, rows_d, rows_r = (rows_ss[sl_i], rows_ds[sl_i],
                                      rows_rs[sl_i])

            def group(g, gcarry):
                def ej(j, sv):
                    e = g * L + j
                    acc0 = jnp.zeros((L,), jnp.float32)
                    acc1 = jnp.zeros((L,), jnp.float32)
                    for r in range(0, D // L, 2):
                        sl = pl.ds(r * L, L)
                        sl1 = pl.ds((r + 1) * L, L)
                        acc0 = acc0 + (rows_s[e, sl] * rows_r[e, sl]
                                       * rows_d[e, sl])
                        acc1 = acc1 + (rows_s[e, sl1] * rows_r[e, sl1]
                                       * rows_d[e, sl1])
                    score = jnp.sum(acc0 + acc1)
                    return jnp.where(lanes == j, score, sv)

                sv = lax.fori_loop(0, L, ej, jnp.zeros((L,), jnp.float32))
                scout[pl.ds(ch * CH1 + g * L, L)] = sv

                dv = dstv[pl.ds(ch * CH1 + g * L, L)]

                def m_body(t, mcarry):
                    cur = plsc.load_gather(smax_local, [dv])
                    nv = jnp.maximum(cur, sv)
                    plsc.store_scatter(smax_local, [dv], nv, mask=lanes == t)
                    return mcarry

                lax.fori_loop(0, L, m_body, 0)
                return gcarry

            lax.fori_loop(0, CH1 // L, group, 0)

        pltpu.sync_copy(scout, scores_hbm.at[pl.ds(soff, SUPE1)])
        return carry

    lax.fori_loop(0, nsup, sup_body, 0)

    pltpu.sync_copy(smax_local, smax_shared.at[s])
    plsc.subcore_barrier()

    def red_init(j, carry):
        red_v[pl.ds(j * L, L)] = neg
        return carry

    lax.fori_loop(0, NPT // L, red_init, 0)
    for t in range(NS):
        pltpu.sync_copy(smax_shared.at[t, pl.ds(s * NPT, NPT)], tmpv)

        def red_b(j, carry):
            sl = pl.ds(j * L, L)
            red_v[sl] = jnp.maximum(red_v[sl], tmpv[sl])
            return carry

        lax.fori_loop(0, NPT // L, red_b, 0)
    pltpu.sync_copy(red_v, pmax_hbm.at[c, pl.ds(s * NPT, NPT)])


def _k1(ent_emb, rel_emb, src, dst, rid):
    return pl.kernel(
        _k1_body,
        out_type=[
            jax.ShapeDtypeStruct((EP,), jnp.float32),
            jax.ShapeDtypeStruct((NC, NPAD), jnp.float32),
        ],
        mesh=_mesh(),
        compiler_params=pltpu.CompilerParams(needs_layout_passes=False),
        scratch_types=[
            pltpu.VMEM((SUPE1,), jnp.int32),
            pltpu.VMEM((SUPE1,), jnp.int32),
            pltpu.VMEM((SUPE1,), jnp.int32),
            pltpu.VMEM((SUPE1,), jnp.float32),
            pltpu.VMEM((CH1,), jnp.int32),
            pltpu.VMEM((CH1,), jnp.int32),
            pltpu.VMEM((CH1,), jnp.int32),
            pltpu.VMEM((CH1,), jnp.int32),
            pltpu.VMEM((CH1,), jnp.int32),
            pltpu.VMEM((CH1,), jnp.int32),
            pltpu.VMEM((CH1, D), jnp.float32),
            pltpu.VMEM((CH1, D), jnp.float32),
            pltpu.VMEM((CH1, D), jnp.float32),
            pltpu.VMEM((CH1, D), jnp.float32),
            pltpu.VMEM((CH1, D), jnp.float32),
            pltpu.VMEM((CH1, D), jnp.float32),
            pltpu.VMEM((NPAD,), jnp.float32),
            pltpu.VMEM((NPT,), jnp.float32),
            pltpu.VMEM((NPT,), jnp.float32),
            pltpu.VMEM_SHARED((NS, NPAD), jnp.float32),
            pltpu.SemaphoreType.DMA,
            pltpu.SemaphoreType.DMA,
        ],
    )(ent_emb, rel_emb, src, dst, rid)


def _k2_body(scores_hbm, dst_hbm, pmax_hbm,
             ex_hbm, pden_hbm,
             p0v, p1v, dstv, dst2v, scv, exv, zv, den_shared, sem):
    c = lax.axis_index("c")
    s = lax.axis_index("s")
    wid = c * NS + s
    base = wid * EPW

    pltpu.async_copy(pmax_hbm.at[0], p0v, sem).wait()
    pltpu.async_copy(pmax_hbm.at[1], p1v, sem).wait()

    zero = jnp.zeros((L,), jnp.float32)

    def z_b(i, carry):
        zv[pl.ds(i * L, L)] = zero
        return carry

    lax.fori_loop(0, NPT // L, z_b, 0)
    pltpu.sync_copy(zv, den_shared.at[pl.ds(s * NPT, NPT)])
    plsc.subcore_barrier()

    def chunk(i, carry):
        off = base + i * CH2
        pltpu.sync_copy(scores_hbm.at[pl.ds(off, CH2)], scv)
        pltpu.sync_copy(dst_hbm.at[pl.ds(off, CH2)], dstv)
        for j in range(CH2 // SB2):
            pltpu.sync_copy(dst_hbm.at[pl.ds(off + j * SB2, SB2)],
                            dst2v.at[j])

        def v_b(k, kcarry):
            sl = pl.ds(k * L, L)
            dv = dstv[sl]
            m0 = plsc.load_gather(p0v, [dv])
            m1 = plsc.load_gather(p1v, [dv])
            exv[sl] = jnp.exp(scv[sl] - jnp.maximum(m0, m1))
            return kcarry

        lax.fori_loop(0, CH2 // L, v_b, 0)
        pltpu.sync_copy(exv, ex_hbm.at[pl.ds(off, CH2)])
        cps = [
            pltpu.async_copy(
                exv.at[pl.ds(j * SB2, SB2)],
                den_shared.at[dst2v.at[j]],
                sem,
                add=True,
            )
            for j in range(CH2 // SB2)
        ]
        for cp in cps:
            cp.wait()
        return carry

    lax.fori_loop(0, EPW // CH2, chunk, 0)
    plsc.subcore_barrier()
    pltpu.sync_copy(den_shared.at[pl.ds(s * NPT, NPT)], zv)
    pltpu.sync_copy(zv, pden_hbm.at[c, pl.ds(s * NPT, NPT)])


def _k2(scores, dst, pmax):
    return pl.kernel(
        _k2_body,
        out_type=[
            jax.ShapeDtypeStruct((EP,), jnp.float32),
            jax.ShapeDtypeStruct((NC, NPAD), jnp.float32),
        ],
        mesh=_mesh(),
        compiler_params=pltpu.CompilerParams(needs_layout_passes=False),
        scratch_types=[
            pltpu.VMEM((NPAD,), jnp.float32),
            pltpu.VMEM((NPAD,), jnp.float32),
            pltpu.VMEM((CH2,), jnp.int32),
            pltpu.VMEM((CH2 // SB2, SB2), jnp.int32),
            pltpu.VMEM((CH2,), jnp.float32),
            pltpu.VMEM((CH2,), jnp.float32),
            pltpu.VMEM((NPT,), jnp.float32),
            pltpu.VMEM_SHARED((NPAD,), jnp.float32),
            pltpu.SemaphoreType.DMA,
        ],
    )(scores, dst, pmax)


SUP = 16
SUPE = SUP * CH4


def _k3_body(ent2_hbm, rel2_hbm, src_hbm, dst_hbm, rid_hbm, ex_hbm,
             pden_hbm,
             nacc_hbm,
             srcv, dstv, ridv, alv, dtotv,
             sf0, sf1, rf0, rf1, dk0, dk1, rows0, rows1, rel0, rel1,
             nacc_shared, sem0, sem1, sem_s0, sem_s1):
    c = lax.axis_index("c")
    s = lax.axis_index("s")
    base = s * EPT
    cN = c * N
    cR = c * R
    sfs, rfs, dks = [sf0, sf1], [rf0, rf1], [dk0, dk1]
    rowss, rels, sems = [rows0, rows1], [rel0, rel1], [sem0, sem1]
    sems_s = [sem_s0, sem_s1]

    pltpu.async_copy(pden_hbm.at[0, pl.ds(0, CH4), :], rows0, sem0).wait()

    def d0_b(r, carry):
        for k in range(DH // L):
            dtotv[pl.ds(r * DH + k * L, L)] = rows0[r, pl.ds(k * L, L)]
        return carry

    lax.fori_loop(0, CH4, d0_b, 0)
    pltpu.async_copy(pden_hbm.at[0, pl.ds(CH4, L), :],
                     rows1.at[pl.ds(0, L), :], sem0).wait()

    def d1_b(r, carry):
        for k in range(DH // L):
            sl = pl.ds((CH4 + r) * DH + k * L, L)
            dtotv[sl] = rows1[r, pl.ds(k * L, L)]
        return carry

    lax.fori_loop(0, L, d1_b, 0)
    pltpu.async_copy(pden_hbm.at[1, pl.ds(0, CH4), :], rows0, sem0).wait()

    def d2_b(r, carry):
        for k in range(DH // L):
            sl = pl.ds(r * DH + k * L, L)
            dtotv[sl] = dtotv[sl] + rows0[r, pl.ds(k * L, L)]
        return carry

    lax.fori_loop(0, CH4, d2_b, 0)
    pltpu.async_copy(pden_hbm.at[1, pl.ds(CH4, L), :],
                     rows1.at[pl.ds(0, L), :], sem0).wait()

    def d3_b(r, carry):
        for k in range(DH // L):
            sl = pl.ds((CH4 + r) * DH + k * L, L)
            dtotv[sl] = dtotv[sl] + rows1[r, pl.ds(k * L, L)]
        return carry

    lax.fori_loop(0, L, d3_b, 0)

    zero = jnp.zeros((L,), jnp.float32)

    def z_b(e, carry):
        for r in range(DH // L):
            rows0[e, pl.ds(r * L, L)] = zero
        return carry

    lax.fori_loop(0, CH4, z_b, 0)
    for b in range(NPT // CH4):
        pltpu.sync_copy(
            rows0, nacc_shared.at[pl.ds(s * NPT + b * CH4, CH4), :]
        )
    plsc.subcore_barrier()

    def build_idx(ch, sl_i):
        for k in range(CH4 // L):
            sl = pl.ds(k * L, L)
            src16 = srcv[pl.ds(ch * CH4 + k * L, L)]
            rid16 = ridv[pl.ds(ch * CH4 + k * L, L)]
            dst16 = dstv[pl.ds(ch * CH4 + k * L, L)]
            sfs[sl_i][sl] = src16 + cN
            rfs[sl_i][sl] = rid16 + cR
            dks[sl_i][sl] = dst16

    def issue(sl_i):
        cp1 = pltpu.async_copy(ent2_hbm.at[sfs[sl_i]], rowss[sl_i],
                               sems[sl_i])
        cp2 = pltpu.async_copy(rel2_hbm.at[rfs[sl_i]], rels[sl_i],
                               sems[sl_i])
        return cp1, cp2

    def sup_body(sup, carry):
        soff = base + sup * SUPE
        pltpu.sync_copy(src_hbm.at[pl.ds(soff, SUPE)], srcv)
        pltpu.sync_copy(dst_hbm.at[pl.ds(soff, SUPE)], dstv)
        pltpu.sync_copy(rid_hbm.at[pl.ds(soff, SUPE)], ridv)
        pltpu.sync_copy(ex_hbm.at[pl.ds(soff, SUPE)],
                        alv.at[pl.ds(0, SUPE)])

        def al_b(k, kcarry):
            sl = pl.ds(k * L, L)
            den = plsc.load_gather(dtotv, [dstv[sl]])
            alv[sl] = alv[sl] / den
            return kcarry

        lax.fori_loop(0, SUPE // L, al_b, 0)

        scat = [None, None]
        for ch in range(SUP):
            sl_i = ch % 2
            if ch == 0:
                build_idx(0, 0)
                cps = issue(0)
            else:
                cps = nxt
            if ch + 1 < SUP:
                s1 = (ch + 1) % 2
                if scat[s1] is not None:
                    scat[s1].wait()
                build_idx(ch + 1, s1)
                nxt = issue(s1)
            cps[0].wait()
            cps[1].wait()
            rows, rel = rowss[sl_i], rels[sl_i]

            def edge(e, ecarry):
                a16 = jnp.full((L,), alv[pl.ds(ch * CH4 + e, L)][0],
                               jnp.float32)
                for r in range(DH // L):
                    sl = pl.ds(r * L, L)
                    rows[e, sl] = rows[e, sl] * rel[e, sl] * a16
                return ecarry

            lax.fori_loop(0, CH4, edge, 0)
            scat[sl_i] = pltpu.async_copy(
                rows, nacc_shared.at[dks[sl_i]], sems_s[sl_i], add=True
            )
        for sc_cp in scat:
            if sc_cp is not None:
                sc_cp.wait()
        return carry

    lax.fori_loop(0, EPT // SUPE, sup_body, 0)
    plsc.subcore_barrier()
    pltpu.sync_copy(
        nacc_shared.at[pl.ds(s * NPT, NPT), :],
        nacc_hbm.at[c, pl.ds(s * NPT, NPT), :],
    )


def _k3(ent2, rel2, src, dst, rid, ex, pden):
    return pl.kernel(
        _k3_body,
        out_type=jax.ShapeDtypeStruct((NC, NPAD, DH), jnp.float32),
        mesh=_mesh(),
        compiler_params=pltpu.CompilerParams(needs_layout_passes=False),
        scratch_types=[
            pltpu.VMEM((SUPE,), jnp.int32),
            pltpu.VMEM((SUPE,), jnp.int32),
            pltpu.VMEM((SUPE,), jnp.int32),
            pltpu.VMEM((SUPE + L,), jnp.float32),
            pltpu.VMEM((NPAD,), jnp.float32),
            pltpu.VMEM((CH4,), jnp.int32),
            pltpu.VMEM((CH4,), jnp.int32),
            pltpu.VMEM((CH4,), jnp.int32),
            pltpu.VMEM((CH4,), jnp.int32),
            pltpu.VMEM((CH4,), jnp.int32),
            pltpu.VMEM((CH4,), jnp.int32),
            pltpu.VMEM((CH4, DH), jnp.float32),
            pltpu.VMEM((CH4, DH), jnp.float32),
            pltpu.VMEM((CH4, DH), jnp.float32),
            pltpu.VMEM((CH4, DH), jnp.float32),
            pltpu.VMEM_SHARED((NPAD, DH), jnp.float32),
            pltpu.SemaphoreType.DMA,
            pltpu.SemaphoreType.DMA,
            pltpu.SemaphoreType.DMA,
            pltpu.SemaphoreType.DMA,
        ],
    )(ent2, rel2, src, dst, rid, ex, pden.reshape(NC, NPAD // DH, DH))


def _mm_body(lo_ref, hi_ref, w_ref, o_ref):
    x = jnp.concatenate([lo_ref[...], hi_ref[...]], axis=1)
    o_ref[...] = jnp.tanh(
        jnp.dot(x, w_ref[...], preferred_element_type=jnp.float32)
    )


def _mm_tanh(lo, hi, w):
    blk = 1024
    return pl.pallas_call(
        _mm_body,
        grid=(NPAD // blk,),
        in_specs=[
            pl.BlockSpec((blk, DH), lambda i: (i, 0)),
            pl.BlockSpec((blk, DH), lambda i: (i, 0)),
            pl.BlockSpec((D, D), lambda i: (0, 0)),
        ],
        out_specs=pl.BlockSpec((blk, D), lambda i: (i, 0)),
        out_shape=jax.ShapeDtypeStruct((NPAD, D), jnp.float32),
    )(lo, hi, w)


def kernel(ent_emb, rel_emb, edge_index, rel_id, neigh_w):
    src = edge_index[0]
    dst = edge_index[1]
    pad = EP - E
    src_p = jnp.concatenate([src, jnp.zeros((pad,), jnp.int32)])
    dst_p = jnp.concatenate([dst, jnp.full((pad,), N, jnp.int32)])
    rid_p = jnp.concatenate([rel_id, jnp.zeros((pad,), jnp.int32)])
    scores, pmax = _k1(ent_emb, rel_emb, src_p, dst_p, rid_p)
    ex, pden = _k2(scores, dst_p, pmax)
    ent2 = ent_emb.reshape(N, 2, DH).swapaxes(0, 1).reshape(2 * N, DH)
    rel2 = rel_emb.reshape(R, 2, DH).swapaxes(0, 1).reshape(2 * R, DH)
    nacc = _k3(ent2, rel2, src_p, dst_p, rid_p, ex, pden)
    return _mm_tanh(nacc[0], nacc[1], neigh_w)[:N]


# speedup vs baseline: 1.0719x; 1.0719x over previous
import jax
import jax.numpy as jnp
from jax import lax
from jax.experimental import pallas as pl
from jax.experimental.pallas import tpu as pltpu
from jax.experimental.pallas import tpu_sc as plsc

E, N, D, R = 160000, 10000, 256, 237
NC, NS, L = 2, 16, 16
NW = NC * NS
NPAD = 10240
NPT = NPAD // NS
EP = 163840
EPW = EP // NW
EPT = EP // NS
CH1 = 64
CH2 = 1024
SB2 = 128
CH4 = 64
DH = D // 2
NEG_INF = float("-inf")


def _mesh():
    return plsc.VectorSubcoreMesh(core_axis_name="c", subcore_axis_name="s")


SUP1 = 8
SUPE1 = SUP1 * CH1
W0T = 6144
W1T = (EP - NS * W0T) // NS


def _k1_body(ent_hbm, rel_hbm, src_hbm, dst_hbm, rid_hbm,
             scores_hbm, pmax_hbm,
             srcv, dstv, ridv, scout,
             ss0, ss1, ds0, ds1, rs0, rs1,
             rowss0, rowss1, rowsd0, rowsd1, rowsr0, rowsr1,
             smax_local, tmpv, red_v, smax_shared, sem0, sem1):
    c = lax.axis_index("c")
    s = lax.axis_index("s")
    base = jnp.where(c == 0, s * W0T, NS * W0T + s * W1T)
    nsup = jnp.where(c == 0, W0T // SUPE1, W1T // SUPE1)
    lanes = lax.iota(jnp.int32, L)
    sss, dss, rss = [ss0, ss1], [ds0, ds1], [rs0, rs1]
    rows_ss, rows_ds, rows_rs = [rowss0, rowss1], [rowsd0, rowsd1], \
        [rowsr0, rowsr1]
    sems = [sem0, sem1]

    neg = jnp.full((L,), NEG_INF, jnp.float32)

    def init_b(i, carry):
        smax_local[pl.ds(i * L, L)] = neg
        return carry

    lax.fori_loop(0, NPAD // L, init_b, 0)

    def build_idx(ch, sl_i):
        for k in range(CH1 // L):
            sl = pl.ds(k * L, L)
            sss[sl_i][sl] = srcv[pl.ds(ch * CH1 + k * L, L)]
            dss[sl_i][sl] = dstv[pl.ds(ch * CH1 + k * L, L)]
            rss[sl_i][sl] = ridv[pl.ds(ch * CH1 + k * L, L)]

    def issue(sl_i):
        cp1 = pltpu.async_copy(ent_hbm.at[sss[sl_i]], rows_ss[sl_i],
                               sems[sl_i])
        cp2 = pltpu.async_copy(ent_hbm.at[dss[sl_i]], rows_ds[sl_i],
                               sems[sl_i])
        cp3 = pltpu.async_copy(rel_hbm.at[rss[sl_i]], rows_rs[sl_i],
                               sems[sl_i])
        return cp1, cp2, cp3

    def sup_body(sup, carry):
        soff = base + sup * SUPE1
        pltpu.sync_copy(src_hbm.at[pl.ds(soff, SUPE1)], srcv)
        pltpu.sync_copy(dst_hbm.at[pl.ds(soff, SUPE1)], dstv)
        pltpu.sync_copy(rid_hbm.at[pl.ds(soff, SUPE1)], ridv)

        for ch in range(SUP1):
            sl_i = ch % 2
            if ch == 0:
                build_idx(0, 0)
                cps = issue(0)
            else:
                cps = nxt
            if ch + 1 < SUP1:
                build_idx(ch + 1, (ch + 1) % 2)
                nxt = issue((ch + 1) % 2)
            cps[0].wait()
            cps[1].wait()
            cps[2].wait()
            rows_s, rows_d, rows_r = (rows_ss[sl_i], rows_ds[sl_i],
                                      rows_rs[sl_i])

            def group(g, gcarry):
                def ej(j, sv):
                    e = g * L + j
                    acc0 = jnp.zeros((L,), jnp.float32)
                    acc1 = jnp.zeros((L,), jnp.float32)
                    for r in range(0, D // L, 2):
                        sl = pl.ds(r * L, L)
                        sl1 = pl.ds((r + 1) * L, L)
                        acc0 = acc0 + (rows_s[e, sl] * rows_r[e, sl]
                                       * rows_d[e, sl])
                        acc1 = acc1 + (rows_s[e, sl1] * rows_r[e, sl1]
                                       * rows_d[e, sl1])
                    score = jnp.sum(acc0 + acc1)
                    return jnp.where(lanes == j, score, sv)

                sv = lax.fori_loop(0, L, ej, jnp.zeros((L,), jnp.float32))
                scout[pl.ds(ch * CH1 + g * L, L)] = sv

                dv = dstv[pl.ds(ch * CH1 + g * L, L)]

                def m_body(t, mcarry):
                    cur = plsc.load_gather(smax_local, [dv])
                    nv = jnp.maximum(cur, sv)
                    plsc.store_scatter(smax_local, [dv], nv, mask=lanes == t)
                    return mcarry

                lax.fori_loop(0, L, m_body, 0)
                return gcarry

            lax.fori_loop(0, CH1 // L, group, 0)

        pltpu.sync_copy(scout, scores_hbm.at[pl.ds(soff, SUPE1)])
        return carry

    lax.fori_loop(0, nsup, sup_body, 0)

    pltpu.sync_copy(smax_local, smax_shared.at[s])
    plsc.subcore_barrier()

    def red_init(j, carry):
        red_v[pl.ds(j * L, L)] = neg
        return carry

    lax.fori_loop(0, NPT // L, red_init, 0)
    for t in range(NS):
        pltpu.sync_copy(smax_shared.at[t, pl.ds(s * NPT, NPT)], tmpv)

        def red_b(j, carry):
            sl = pl.ds(j * L, L)
            red_v[sl] = jnp.maximum(red_v[sl], tmpv[sl])
            return carry

        lax.fori_loop(0, NPT // L, red_b, 0)
    pltpu.sync_copy(red_v, pmax_hbm.at[c, pl.ds(s * NPT, NPT)])


def _k1(ent_emb, rel_emb, src, dst, rid):
    return pl.kernel(
        _k1_body,
        out_type=[
            jax.ShapeDtypeStruct((EP,), jnp.float32),
            jax.ShapeDtypeStruct((NC, NPAD), jnp.float32),
        ],
        mesh=_mesh(),
        compiler_params=pltpu.CompilerParams(needs_layout_passes=False),
        scratch_types=[
            pltpu.VMEM((SUPE1,), jnp.int32),
            pltpu.VMEM((SUPE1,), jnp.int32),
            pltpu.VMEM((SUPE1,), jnp.int32),
            pltpu.VMEM((SUPE1,), jnp.float32),
            pltpu.VMEM((CH1,), jnp.int32),
            pltpu.VMEM((CH1,), jnp.int32),
            pltpu.VMEM((CH1,), jnp.int32),
            pltpu.VMEM((CH1,), jnp.int32),
            pltpu.VMEM((CH1,), jnp.int32),
            pltpu.VMEM((CH1,), jnp.int32),
            pltpu.VMEM((CH1, D), jnp.float32),
            pltpu.VMEM((CH1, D), jnp.float32),
            pltpu.VMEM((CH1, D), jnp.float32),
            pltpu.VMEM((CH1, D), jnp.float32),
            pltpu.VMEM((CH1, D), jnp.float32),
            pltpu.VMEM((CH1, D), jnp.float32),
            pltpu.VMEM((NPAD,), jnp.float32),
            pltpu.VMEM((NPT,), jnp.float32),
            pltpu.VMEM((NPT,), jnp.float32),
            pltpu.VMEM_SHARED((NS, NPAD), jnp.float32),
            pltpu.SemaphoreType.DMA,
            pltpu.SemaphoreType.DMA,
        ],
    )(ent_emb, rel_emb, src, dst, rid)


def _k2_body(scores_hbm, dst_hbm, pmax_hbm,
             ex_hbm, pden_hbm,
             p0v, p1v, dstv, dst2v, scv, exv, zv, den_shared, sem):
    c = lax.axis_index("c")
    s = lax.axis_index("s")
    wid = c * NS + s
    base = wid * EPW

    pltpu.async_copy(pmax_hbm.at[0], p0v, sem).wait()
    pltpu.async_copy(pmax_hbm.at[1], p1v, sem).wait()

    zero = jnp.zeros((L,), jnp.float32)

    def z_b(i, carry):
        zv[pl.ds(i * L, L)] = zero
        return carry

    lax.fori_loop(0, NPT // L, z_b, 0)
    pltpu.sync_copy(zv, den_shared.at[pl.ds(s * NPT, NPT)])
    plsc.subcore_barrier()

    def chunk(i, carry):
        off = base + i * CH2
        pltpu.sync_copy(scores_hbm.at[pl.ds(off, CH2)], scv)
        pltpu.sync_copy(dst_hbm.at[pl.ds(off, CH2)], dstv)
        for j in range(CH2 // SB2):
            pltpu.sync_copy(dst_hbm.at[pl.ds(off + j * SB2, SB2)],
                            dst2v.at[j])

        def v_b(k, kcarry):
            sl = pl.ds(k * L, L)
            dv = dstv[sl]
            m0 = plsc.load_gather(p0v, [dv])
            m1 = plsc.load_gather(p1v, [dv])
            exv[sl] = jnp.exp(scv[sl] - jnp.maximum(m0, m1))
            return kcarry

        lax.fori_loop(0, CH2 // L, v_b, 0)
        pltpu.sync_copy(exv, ex_hbm.at[pl.ds(off, CH2)])
        cps = [
            pltpu.async_copy(
                exv.at[pl.ds(j * SB2, SB2)],
                den_shared.at[dst2v.at[j]],
                sem,
                add=True,
            )
            for j in range(CH2 // SB2)
        ]
        for cp in cps:
            cp.wait()
        return carry

    lax.fori_loop(0, EPW // CH2, chunk, 0)
    plsc.subcore_barrier()
    pltpu.sync_copy(den_shared.at[pl.ds(s * NPT, NPT)], zv)
    pltpu.sync_copy(zv, pden_hbm.at[c, pl.ds(s * NPT, NPT)])


def _k2(scores, dst, pmax):
    return pl.kernel(
        _k2_body,
        out_type=[
            jax.ShapeDtypeStruct((EP,), jnp.float32),
            jax.ShapeDtypeStruct((NC, NPAD), jnp.float32),
        ],
        mesh=_mesh(),
        compiler_params=pltpu.CompilerParams(needs_layout_passes=False),
        scratch_types=[
            pltpu.VMEM((NPAD,), jnp.float32),
            pltpu.VMEM((NPAD,), jnp.float32),
            pltpu.VMEM((CH2,), jnp.int32),
            pltpu.VMEM((CH2 // SB2, SB2), jnp.int32),
            pltpu.VMEM((CH2,), jnp.float32),
            pltpu.VMEM((CH2,), jnp.float32),
            pltpu.VMEM((NPT,), jnp.float32),
            pltpu.VMEM_SHARED((NPAD,), jnp.float32),
            pltpu.SemaphoreType.DMA,
        ],
    )(scores, dst, pmax)


SUP = 16
SUPE = SUP * CH4


def _k3_body(ent2_hbm, rel2_hbm, src_hbm, dst_hbm, rid_hbm, ex_hbm,
             pden_hbm,
             nacc_hbm,
             srcv, dstv, ridv, alv, dtotv,
             sf0, sf1, rf0, rf1, dk0, dk1, rows0, rows1, rel0, rel1,
             nacc_shared, sem0, sem1, sem_s0, sem_s1):
    c = lax.axis_index("c")
    s = lax.axis_index("s")
    base = s * EPT
    cN = c * N
    cR = c * R
    sfs, rfs, dks = [sf0, sf1], [rf0, rf1], [dk0, dk1]
    rowss, rels, sems = [rows0, rows1], [rel0, rel1], [sem0, sem1]
    sems_s = [sem_s0, sem_s1]

    pltpu.async_copy(pden_hbm.at[0, pl.ds(0, CH4), :], rows0, sem0).wait()

    def d0_b(r, carry):
        for k in range(DH // L):
            dtotv[pl.ds(r * DH + k * L, L)] = rows0[r, pl.ds(k * L, L)]
        return carry

    lax.fori_loop(0, CH4, d0_b, 0)
    pltpu.async_copy(pden_hbm.at[0, pl.ds(CH4, L), :],
                     rows1.at[pl.ds(0, L), :], sem0).wait()

    def d1_b(r, carry):
        for k in range(DH // L):
            sl = pl.ds((CH4 + r) * DH + k * L, L)
            dtotv[sl] = rows1[r, pl.ds(k * L, L)]
        return carry

    lax.fori_loop(0, L, d1_b, 0)
    pltpu.async_copy(pden_hbm.at[1, pl.ds(0, CH4), :], rows0, sem0).wait()

    def d2_b(r, carry):
        for k in range(DH // L):
            sl = pl.ds(r * DH + k * L, L)
            dtotv[sl] = dtotv[sl] + rows0[r, pl.ds(k * L, L)]
        return carry

    lax.fori_loop(0, CH4, d2_b, 0)
    pltpu.async_copy(pden_hbm.at[1, pl.ds(CH4, L), :],
                     rows1.at[pl.ds(0, L), :], sem0).wait()

    def d3_b(r, carry):
        for k in range(DH // L):
            sl = pl.ds((CH4 + r) * DH + k * L, L)
            dtotv[sl] = dtotv[sl] + rows1[r, pl.ds(k * L, L)]
        return carry

    lax.fori_loop(0, L, d3_b, 0)

    zero = jnp.zeros((L,), jnp.float32)

    def z_b(e, carry):
        for r in range(DH // L):
            rows0[e, pl.ds(r * L, L)] = zero
        return carry

    lax.fori_loop(0, CH4, z_b, 0)
    for b in range(NPT // CH4):
        pltpu.sync_copy(
            rows0, nacc_shared.at[pl.ds(s * NPT + b * CH4, CH4), :]
        )
    plsc.subcore_barrier()

    def build_idx(ch, sl_i):
        for k in range(CH4 // L):
            sl = pl.ds(k * L, L)
            src16 = srcv[pl.ds(ch * CH4 + k * L, L)]
            rid16 = ridv[pl.ds(ch * CH4 + k * L, L)]
            dst16 = dstv[pl.ds(ch * CH4 + k * L, L)]
            sfs[sl_i][sl] = src16 + cN
            rfs[sl_i][sl] = rid16 + cR
            dks[sl_i][sl] = dst16

    def issue(sl_i):
        cp1 = pltpu.async_copy(ent2_hbm.at[sfs[sl_i]], rowss[sl_i],
                               sems[sl_i])
        cp2 = pltpu.async_copy(rel2_hbm.at[rfs[sl_i]], rels[sl_i],
                               sems[sl_i])
        return cp1, cp2

    def sup_body(sup, carry):
        soff = base + sup * SUPE
        pltpu.sync_copy(src_hbm.at[pl.ds(soff, SUPE)], srcv)
        pltpu.sync_copy(dst_hbm.at[pl.ds(soff, SUPE)], dstv)
        pltpu.sync_copy(rid_hbm.at[pl.ds(soff, SUPE)], ridv)
        pltpu.sync_copy(ex_hbm.at[pl.ds(soff, SUPE)],
                        alv.at[pl.ds(0, SUPE)])

        def al_b(k, kcarry):
            sl = pl.ds(k * L, L)
            den = plsc.load_gather(dtotv, [dstv[sl]])
            alv[sl] = alv[sl] / den
            return kcarry

        lax.fori_loop(0, SUPE // L, al_b, 0)

        scat = [None, None]
        for ch in range(SUP):
            sl_i = ch % 2
            if ch == 0:
                build_idx(0, 0)
                cps = issue(0)
            else:
                cps = nxt
            if ch + 1 < SUP:
                s1 = (ch + 1) % 2
                if scat[s1] is not None:
                    scat[s1].wait()
                build_idx(ch + 1, s1)
                nxt = issue(s1)
            cps[0].wait()
            cps[1].wait()
            rows, rel = rowss[sl_i], rels[sl_i]

            def edge(e, ecarry):
                a16 = jnp.full((L,), alv[pl.ds(ch * CH4 + e, L)][0],
                               jnp.float32)
                for r in range(DH // L):
                    sl = pl.ds(r * L, L)
                    rows[e, sl] = rows[e, sl] * rel[e, sl] * a16
                return ecarry

            lax.fori_loop(0, CH4, edge, 0)
            scat[sl_i] = pltpu.async_copy(
                rows, nacc_shared.at[dks[sl_i]], sems_s[sl_i], add=True
            )
        for sc_cp in scat:
            if sc_cp is not None:
                sc_cp.wait()
        return carry

    lax.fori_loop(0, EPT // SUPE, sup_body, 0)
    plsc.subcore_barrier()
    pltpu.sync_copy(
        nacc_shared.at[pl.ds(s * NPT, NPT), :],
        nacc_hbm.at[c, pl.ds(s * NPT, NPT), :],
    )


def _k3(ent2, rel2, src, dst, rid, ex, pden):
    return pl.kernel(
        _k3_body,
        out_type=jax.ShapeDtypeStruct((NC, NPAD, DH), jnp.float32),
        mesh=_mesh(),
        compiler_params=pltpu.CompilerParams(needs_layout_passes=False),
        scratch_types=[
            pltpu.VMEM((SUPE,), jnp.int32),
            pltpu.VMEM((SUPE,), jnp.int32),
            pltpu.VMEM((SUPE,), jnp.int32),
            pltpu.VMEM((SUPE + L,), jnp.float32),
            pltpu.VMEM((NPAD,), jnp.float32),
            pltpu.VMEM((CH4,), jnp.int32),
            pltpu.VMEM((CH4,), jnp.int32),
            pltpu.VMEM((CH4,), jnp.int32),
            pltpu.VMEM((CH4,), jnp.int32),
            pltpu.VMEM((CH4,), jnp.int32),
            pltpu.VMEM((CH4,), jnp.int32),
            pltpu.VMEM((CH4, DH), jnp.float32),
            pltpu.VMEM((CH4, DH), jnp.float32),
            pltpu.VMEM((CH4, DH), jnp.float32),
            pltpu.VMEM((CH4, DH), jnp.float32),
            pltpu.VMEM_SHARED((NPAD, DH), jnp.float32),
            pltpu.SemaphoreType.DMA,
            pltpu.SemaphoreType.DMA,
            pltpu.SemaphoreType.DMA,
            pltpu.SemaphoreType.DMA,
        ],
    )(ent2, rel2, src, dst, rid, ex, pden.reshape(NC, NPAD // DH, DH))


def _mm_body(lo_ref, hi_ref, w_ref, o_ref):
    x = jnp.concatenate([lo_ref[...], hi_ref[...]], axis=1)
    o_ref[...] = jnp.tanh(
        jnp.dot(x, w_ref[...], preferred_element_type=jnp.float32)
    )


def _mm_tanh(lo, hi, w):
    blk = 1024
    return pl.pallas_call(
        _mm_body,
        grid=(NPAD // blk,),
        in_specs=[
            pl.BlockSpec((blk, DH), lambda i: (i, 0)),
            pl.BlockSpec((blk, DH), lambda i: (i, 0)),
            pl.BlockSpec((D, D), lambda i: (0, 0)),
        ],
        out_specs=pl.BlockSpec((blk, D), lambda i: (i, 0)),
        out_shape=jax.ShapeDtypeStruct((NPAD, D), jnp.float32),
    )(lo, hi, w)


def kernel(ent_emb, rel_emb, edge_index, rel_id, neigh_w):
    src = edge_index[0]
    dst = edge_index[1]
    pad = EP - E
    src_p = jnp.concatenate([src, jnp.zeros((pad,), jnp.int32)])
    dst_p = jnp.concatenate([dst, jnp.full((pad,), N, jnp.int32)])
    rid_p = jnp.concatenate([rel_id, jnp.zeros((pad,), jnp.int32)])
    scores, pmax = _k1(ent_emb, rel_emb, src_p, dst_p, rid_p)
    ex, pden = _k2(scores, dst_p, pmax)
    ent2 = ent_emb.reshape(N, 2, DH).swapaxes(0, 1).reshape(2 * N, DH)
    rel2 = rel_emb.reshape(R, 2, DH).swapaxes(0, 1).reshape(2 * R, DH)
    nacc = _k3(ent2, rel2, src_p, dst_p, rid_p, ex, pden)
    return _mm_tanh(nacc[0], nacc[1], neigh_w)[:N]
